# baseline (speedup 1.0000x reference)
import functools

import jax
import jax.numpy as jnp
from jax import lax
from jax.experimental import pallas as pl
from jax.experimental.pallas import tpu as pltpu
from jax.experimental.pallas import tpu_sc as plsc

B = 16384
D = 32
NC = 2
NS = 16
NW = NC * NS
BPW = B // NW
CHUNK = 128
NCHUNK = BPW // CHUNK
ROWS_PER_LINE = 128 // D


def _sc_body(uidx_hbm, iidx_hbm, uln_hbm, iln_hbm, ubf_hbm, ibf_hbm,
             partials_hbm, bias_hbm,
             uidx_v, iidx_v, ugid_v, igid_v, ugrp_v, igrp_v,
             ubv_v, ibv_v, acc_v, outb_v,
             sem_u, sem_i, sem_ub, sem_ib):
    wid = lax.axis_index("s") * NC + lax.axis_index("c")
    base = wid * BPW
    row0 = wid * NCHUNK

    pltpu.sync_copy(uidx_hbm.at[pl.ds(row0, NCHUNK)], uidx_v)
    pltpu.sync_copy(iidx_hbm.at[pl.ds(row0, NCHUNK)], iidx_v)

    bias_copies = []
    for j in range(NCHUNK):
        bias_copies.append(pltpu.async_copy(
            ubf_hbm.at[uidx_v.at[j]], ubv_v.at[pl.ds(j * CHUNK, CHUNK)],
            sem_ub))
        bias_copies.append(pltpu.async_copy(
            ibf_hbm.at[iidx_v.at[j]], ibv_v.at[pl.ds(j * CHUNK, CHUNK)],
            sem_ib))

    for j in range(NCHUNK):
        for k in range(CHUNK // 16):
            u = uidx_v[j, pl.ds(k * 16, 16)]
            i = iidx_v[j, pl.ds(k * 16, 16)]
            ugid_v[j, pl.ds(k * 16, 16)] = lax.shift_right_logical(u, 2)
            igid_v[j, pl.ds(k * 16, 16)] = lax.shift_right_logical(i, 2)

    lanes = lax.iota(jnp.int32, 16)
    acc = jnp.zeros((16,), jnp.float32)

    for j in range(NCHUNK):
        cu = pltpu.async_copy(uln_hbm.at[ugid_v.at[j]], ugrp_v, sem_u)
        ci = pltpu.async_copy(iln_hbm.at[igid_v.at[j]], igrp_v, sem_i)
        cu.wait()
        ci.wait()
        for k in range(CHUNK // 16):
            uoff = (uidx_v[j, pl.ds(k * 16, 16)] & 3) * D
            ioff = (iidx_v[j, pl.ds(k * 16, 16)] & 3) * D
            rows = lanes + k * 16
            for d in range(D):
                ud = plsc.load_gather(ugrp_v, [rows, uoff + d])
                vd = plsc.load_gather(igrp_v, [rows, ioff + d])
                acc = acc + ud * vd

    acc_v[...] = acc
    pltpu.sync_copy(acc_v, partials_hbm.at[wid])

    for c in bias_copies:
        c.wait()
    for j in range(BPW // 16):
        outb_v[pl.ds(j * 16, 16)] = (
            ubv_v[pl.ds(j * 16, 16)] + ibv_v[pl.ds(j * 16, 16)])
    pltpu.sync_copy(outb_v, bias_hbm.at[pl.ds(base, BPW)])


@functools.cache
def _make_sc_call():
    return pl.kernel(
        _sc_body,
        out_type=[
            jax.ShapeDtypeStruct((NW, 16), jnp.float32),
            jax.ShapeDtypeStruct((B,), jnp.float32),
        ],
        mesh=plsc.VectorSubcoreMesh(
            core_axis_name="c", subcore_axis_name="s"),
        compiler_params=pltpu.CompilerParams(
            use_tc_tiling_on_sc=True, needs_layout_passes=False),
        scratch_types=[
            pltpu.VMEM((NCHUNK, CHUNK), jnp.int32),
            pltpu.VMEM((NCHUNK, CHUNK), jnp.int32),
            pltpu.VMEM((NCHUNK, CHUNK), jnp.int32),
            pltpu.VMEM((NCHUNK, CHUNK), jnp.int32),
            pltpu.VMEM((CHUNK, 128), jnp.float32),
            pltpu.VMEM((CHUNK, 128), jnp.float32),
            pltpu.VMEM((BPW,), jnp.float32),
            pltpu.VMEM((BPW,), jnp.float32),
            pltpu.VMEM((16,), jnp.float32),
            pltpu.VMEM((BPW,), jnp.float32),
            pltpu.SemaphoreType.DMA,
            pltpu.SemaphoreType.DMA,
            pltpu.SemaphoreType.DMA,
            pltpu.SemaphoreType.DMA,
        ],
    )


def _tc_body(bias_ref, partials_ref, out_ref):
    total = jnp.sum(partials_ref[...])
    out_ref[...] = bias_ref[...] + total


_tc_call = pl.pallas_call(
    _tc_body,
    out_shape=jax.ShapeDtypeStruct((128, 128), jnp.float32),
)


def kernel(user_index, item_index, user_emb, item_emb, ub, ib):
    uidx2d = user_index.astype(jnp.int32).reshape(B // CHUNK, CHUNK)
    iidx2d = item_index.astype(jnp.int32).reshape(B // CHUNK, CHUNK)
    uln = user_emb.reshape(-1, 128)
    iln = item_emb.reshape(-1, 128)
    ubf = ub.reshape(-1)
    ibf = ib.reshape(-1)
    partials, bias = _make_sc_call()(uidx2d, iidx2d, uln, iln, ubf, ibf)
    out2d = _tc_call(bias.reshape(128, 128), partials)
    return out2d.reshape(B)


# speedup vs baseline: 1.0114x; 1.0114x over previous
import functools

import jax
import jax.numpy as jnp
from jax import lax
from jax.experimental import pallas as pl
from jax.experimental.pallas import tpu as pltpu
from jax.experimental.pallas import tpu_sc as plsc

B = 16384
D = 32
NC = 2
NS = 16
NW = NC * NS
BPW = B // NW
CHUNK = 128
NCHUNK = BPW // CHUNK


def _sc_body(uidx_hbm, iidx_hbm, uln_hbm, iln_hbm, ubf_hbm, ibf_hbm,
             partials_hbm, bias_hbm,
             uidx_v, iidx_v, ugrp_v, igrp_v,
             ubv_v, ibv_v, acc_v, outb_v,
             sem_u, sem_i, sem_ub, sem_ib):
    wid = lax.axis_index("s") * NC + lax.axis_index("c")
    base = wid * BPW
    row0 = wid * NCHUNK

    pltpu.sync_copy(uidx_hbm.at[pl.ds(row0, NCHUNK)], uidx_v)
    pltpu.sync_copy(iidx_hbm.at[pl.ds(row0, NCHUNK)], iidx_v)

    bias_copies = []
    for j in range(NCHUNK):
        bias_copies.append(pltpu.async_copy(
            ubf_hbm.at[uidx_v.at[j]], ubv_v.at[pl.ds(j * CHUNK, CHUNK)],
            sem_ub))
        bias_copies.append(pltpu.async_copy(
            ibf_hbm.at[iidx_v.at[j]], ibv_v.at[pl.ds(j * CHUNK, CHUNK)],
            sem_ib))

    acc = jnp.zeros((16,), jnp.float32)

    for j in range(NCHUNK):
        cu = pltpu.async_copy(uln_hbm.at[uidx_v.at[j]], ugrp_v, sem_u)
        ci = pltpu.async_copy(iln_hbm.at[iidx_v.at[j]], igrp_v, sem_i)
        cu.wait()
        ci.wait()

        def dot_body(n, a):
            u0 = ugrp_v[n, pl.ds(0, 16)]
            v0 = igrp_v[n, pl.ds(0, 16)]
            u1 = ugrp_v[n, pl.ds(16, 16)]
            v1 = igrp_v[n, pl.ds(16, 16)]
            return a + u0 * v0 + u1 * v1

        acc = lax.fori_loop(0, CHUNK, dot_body, acc, unroll=4)

    acc_v[...] = acc
    pltpu.sync_copy(acc_v, partials_hbm.at[wid])

    for c in bias_copies:
        c.wait()
    for j in range(BPW // 16):
        outb_v[pl.ds(j * 16, 16)] = (
            ubv_v[pl.ds(j * 16, 16)] + ibv_v[pl.ds(j * 16, 16)])
    pltpu.sync_copy(outb_v, bias_hbm.at[pl.ds(base, BPW)])


@functools.cache
def _make_sc_call():
    return pl.kernel(
        _sc_body,
        out_type=[
            jax.ShapeDtypeStruct((NW, 16), jnp.float32),
            jax.ShapeDtypeStruct((B,), jnp.float32),
        ],
        mesh=plsc.VectorSubcoreMesh(
            core_axis_name="c", subcore_axis_name="s"),
        compiler_params=pltpu.CompilerParams(
            use_tc_tiling_on_sc=True, needs_layout_passes=False),
        scratch_types=[
            pltpu.VMEM((NCHUNK, CHUNK), jnp.int32),
            pltpu.VMEM((NCHUNK, CHUNK), jnp.int32),
            pltpu.VMEM((CHUNK, 128), jnp.float32),
            pltpu.VMEM((CHUNK, 128), jnp.float32),
            pltpu.VMEM((BPW,), jnp.float32),
            pltpu.VMEM((BPW,), jnp.float32),
            pltpu.VMEM((16,), jnp.float32),
            pltpu.VMEM((BPW,), jnp.float32),
            pltpu.SemaphoreType.DMA,
            pltpu.SemaphoreType.DMA,
            pltpu.SemaphoreType.DMA,
            pltpu.SemaphoreType.DMA,
        ],
    )


def _tc_body(bias_ref, partials_ref, out_ref):
    total = jnp.sum(partials_ref[...])
    out_ref[...] = bias_ref[...] + total


_tc_call = pl.pallas_call(
    _tc_body,
    out_shape=jax.ShapeDtypeStruct((128, 128), jnp.float32),
)


def kernel(user_index, item_index, user_emb, item_emb, ub, ib):
    uidx2d = user_index.astype(jnp.int32).reshape(B // CHUNK, CHUNK)
    iidx2d = item_index.astype(jnp.int32).reshape(B // CHUNK, CHUNK)
    uln = jnp.pad(user_emb, ((0, 0), (0, 128 - D)))
    iln = jnp.pad(item_emb, ((0, 0), (0, 128 - D)))
    ubf = ub.reshape(-1)
    ibf = ib.reshape(-1)
    partials, bias = _make_sc_call()(uidx2d, iidx2d, uln, iln, ubf, ibf)
    out2d = _tc_call(bias.reshape(128, 128), partials)
    return out2d.reshape(B)
